# baseline (speedup 1.0000x reference)
import functools
import jax
import jax.numpy as jnp
from jax import lax
from jax.experimental import pallas as pl
from jax.experimental.pallas import tpu as pltpu
from jax.experimental.pallas import tpu_sc as plsc

N = 10000
E = 320000
D = 128
H = 4
HID = 128
C = 40

ETOT = E + N
EPAD = 331776
CH = 16
NT = 16
NPAD = 10240
ROWS_PER_TILE = NPAD // NT

L1_CHUNKS = EPAD // (NT * CH)
L2_CHUNKS = EPAD // (2 * NT * CH)

_mesh = plsc.VectorSubcoreMesh(core_axis_name="c", subcore_axis_name="s")


def _elu(v):
    return jnp.where(v > 0, v, jnp.exp(v) - 1.0)


def _tc1_body(x_ref, w_ref, b_ref, o_ref):
    v = jnp.dot(x_ref[...], w_ref[...], preferred_element_type=jnp.float32)
    o_ref[...] = _elu(v + b_ref[...][None, :])


def _input_proj(x, W_in, b_in):
    blk = 1000
    return pl.pallas_call(
        _tc1_body,
        grid=(N // blk,),
        in_specs=[
            pl.BlockSpec((blk, D), lambda i: (i, 0)),
            pl.BlockSpec((D, HID), lambda i: (0, 0)),
            pl.BlockSpec((HID,), lambda i: (0,)),
        ],
        out_specs=pl.BlockSpec((blk, HID), lambda i: (i, 0)),
        out_shape=jax.ShapeDtypeStruct((N, HID), jnp.float32),
    )(x, W_in, b_in)


def _tc2_body(h_ref, wl_ref, wr_ref, ol_ref, or_ref):
    blk = h_ref[...]
    ol_ref[0] = jnp.dot(blk, wl_ref[...], preferred_element_type=jnp.float32)
    or_ref[0] = jnp.dot(blk, wr_ref[...], preferred_element_type=jnp.float32)


def _proj_l1(h0, Wl1, Wr1):
    blk = 1000
    return pl.pallas_call(
        _tc2_body,
        grid=(N // blk, H),
        in_specs=[
            pl.BlockSpec((blk, HID), lambda i, h: (i, 0)),
            pl.BlockSpec((HID, HID), lambda i, h: (0, h)),
            pl.BlockSpec((HID, HID), lambda i, h: (0, h)),
        ],
        out_specs=[
            pl.BlockSpec((1, blk, HID), lambda i, h: (h, i, 0)),
            pl.BlockSpec((1, blk, HID), lambda i, h: (h, i, 0)),
        ],
        out_shape=[
            jax.ShapeDtypeStruct((H, N, HID), jnp.float32),
            jax.ShapeDtypeStruct((H, N, HID), jnp.float32),
        ],
    )(h0, Wl1, Wr1)


def _tc3_body(raw_ref, den_ref, b_ref, wl_ref, wr_ref, ol_ref, or_ref):
    h = pl.program_id(1)

    @pl.when(h == 0)
    def _():
        ol_ref[...] = jnp.zeros_like(ol_ref)
        or_ref[...] = jnp.zeros_like(or_ref)

    den = den_ref[0, :, 0:1] + 1e-16
    hp = _elu(raw_ref[0] / den + b_ref[0])
    ol_ref[...] += jnp.dot(hp, wl_ref[0], preferred_element_type=jnp.float32)
    or_ref[...] += jnp.dot(hp, wr_ref[0], preferred_element_type=jnp.float32)


def _proj_l2(out1, den1, bias1, Wl2, Wr2):
    blk = 1000
    return pl.pallas_call(
        _tc3_body,
        grid=(N // blk, H),
        in_specs=[
            pl.BlockSpec((1, blk, HID), lambda i, h: (h, i, 0)),
            pl.BlockSpec((1, blk, 16), lambda i, h: (h, i, 0)),
            pl.BlockSpec((1, 1, HID), lambda i, h: (h, 0, 0)),
            pl.BlockSpec((1, HID, HID), lambda i, h: (h, 0, 0)),
            pl.BlockSpec((1, HID, HID), lambda i, h: (h, 0, 0)),
        ],
        out_specs=[
            pl.BlockSpec((blk, HID), lambda i, h: (i, 0)),
            pl.BlockSpec((blk, HID), lambda i, h: (i, 0)),
        ],
        out_shape=[
            jax.ShapeDtypeStruct((N, HID), jnp.float32),
            jax.ShapeDtypeStruct((N, HID), jnp.float32),
        ],
    )(out1, den1, bias1.reshape(H, 1, HID), Wl2.reshape(H, HID, HID),
      Wr2.reshape(H, HID, HID))


def _tc4_body(raw_ref, den_ref, b2_ref, wo_ref, bo_ref, o_ref):
    raw = raw_ref[0] + raw_ref[1]
    den = den_ref[0, :, 0:1] + den_ref[1, :, 0:1] + 1e-16
    h2 = _elu(raw / den + b2_ref[...][None, :])
    logits = jnp.dot(h2, wo_ref[...], preferred_element_type=jnp.float32)
    logits = logits + bo_ref[...][None, :]
    m = jnp.max(logits, axis=1, keepdims=True)
    lse = m + jnp.log(jnp.sum(jnp.exp(logits - m), axis=1, keepdims=True))
    o_ref[...] = logits - lse


def _output_head(out2, den2, bias2, W_out, b_out):
    blk = 1000
    return pl.pallas_call(
        _tc4_body,
        grid=(N // blk,),
        in_specs=[
            pl.BlockSpec((2, blk, HID), lambda i: (0, i, 0)),
            pl.BlockSpec((2, blk, 16), lambda i: (0, i, 0)),
            pl.BlockSpec((HID,), lambda i: (0,)),
            pl.BlockSpec((HID, C), lambda i: (0, 0)),
            pl.BlockSpec((C,), lambda i: (0,)),
        ],
        out_specs=pl.BlockSpec((blk, C), lambda i: (i, 0)),
        out_shape=jax.ShapeDtypeStruct((N, C), jnp.float32),
    )(out2, den2, bias2, W_out, b_out)


ZR = 16


def _zero_scratch(zbuf, dzbuf):
    def zrow(i, _):
        for j in range(8):
            zbuf[i, pl.ds(j * 16, 16)] = jnp.zeros((16,), jnp.float32)
        dzbuf[i, :] = jnp.zeros((16,), jnp.float32)
        return 0

    lax.fori_loop(0, ZR, zrow, 0)


def _zero_acc(sid, acc, dacc, zbuf, dzbuf):
    base = sid * ROWS_PER_TILE
    for k in range(ROWS_PER_TILE // ZR):
        pltpu.sync_copy(zbuf, acc.at[pl.ds(base + k * ZR, ZR)])
        pltpu.sync_copy(dzbuf, dacc.at[pl.ds(base + k * ZR, ZR)])


def _edge_chunk(bufL, bufR, attv, msg, den, ebase):
    lane = lax.iota(jnp.int32, 16)
    att = [attv[pl.ds(j * 16, 16)] for j in range(8)]
    for g in range(CH // 16):
        alv = jnp.zeros((16,), jnp.float32)
        for el in range(16):
            e = g * 16 + el
            acc_a = jnp.zeros((16,), jnp.float32)
            for j in range(8):
                sl = pl.ds(j * 16, 16)
                a = bufL[e, sl] + bufR[e, sl]
                a = jnp.maximum(a, 0.2 * a)
                acc_a = acc_a + a * att[j]
            alpha_e = plsc.cumsum(acc_a)[15]
            alv = jnp.where(lane == el, alpha_e, alv)
        exv = jnp.exp(alv)
        exv = jnp.where(ebase + g * 16 + lane < ETOT, exv, 0.0)
        for el in range(16):
            e = g * 16 + el
            exe = exv[el]
            for j in range(8):
                sl = pl.ds(j * 16, 16)
                msg[e, sl] = exe * bufL[e, sl]
            den[e, :] = jnp.where(lane == 0, exe, 0.0)


def _edge_loop(xl_tab, xr_tab, nchunks, ebase0,
               sv, tv, bufL, bufR, msg, den, attv, semL, semR, semS, semD,
               acc, dacc):
    def issue(ci, b):
        pltpu.async_copy(xl_tab.at[sv.at[ci]], bufL.at[b], semL.at[b])
        pltpu.async_copy(xr_tab.at[tv.at[ci]], bufR.at[b], semR.at[b])

    def wait_scatter(b):
        pltpu.make_async_copy(msg.at[b], acc.at[tv.at[0]], semS.at[b]).wait()
        pltpu.make_async_copy(den.at[b], dacc.at[tv.at[0]], semD.at[b]).wait()

    issue(0, 0)

    def pair(ci2, _):
        for b in range(2):
            ci = ci2 * 2 + b
            pltpu.make_async_copy(xl_tab.at[sv.at[0]], bufL.at[b], semL.at[b]).wait()
            pltpu.make_async_copy(xr_tab.at[tv.at[0]], bufR.at[b], semR.at[b]).wait()

            @pl.when(ci + 1 < nchunks)
            def _():
                issue(ci + 1, 1 - b)

            @pl.when(ci2 > 0)
            def _():
                wait_scatter(b)

            _edge_chunk(bufL.at[b], bufR.at[b], attv, msg.at[b], den.at[b],
                        ebase0 + ci * CH)
            pltpu.async_copy(msg.at[b], acc.at[tv.at[ci]], semS.at[b], add=True)
            pltpu.async_copy(den.at[b], dacc.at[tv.at[ci]], semD.at[b], add=True)
        return 0

    lax.fori_loop(0, nchunks // 2, pair, 0)
    wait_scatter(0)
    wait_scatter(1)


def _l1_body(xl_hbm, xr_hbm, s3_hbm, t3_hbm, att_hbm,
             out_hbm, dout_hbm,
             sv, tv, bufL, bufR, msg, den, attv, zbuf, dzbuf,
             semL, semR, semS, semD, acc, dacc):
    cid = lax.axis_index("c")
    sid = lax.axis_index("s")
    _zero_scratch(zbuf, dzbuf)
    hseg_chunks = L1_CHUNKS // 4
    for hp in range(2):
        h = cid * 2 + hp
        pltpu.sync_copy(att_hbm.at[h], attv)
        _zero_acc(sid, acc, dacc, zbuf, dzbuf)
        plsc.subcore_barrier()

        def hseg_body(hseg, _):
            pltpu.sync_copy(s3_hbm.at[sid, pl.ds(hseg * hseg_chunks, hseg_chunks)], sv)
            pltpu.sync_copy(t3_hbm.at[sid, pl.ds(hseg * hseg_chunks, hseg_chunks)], tv)
            ebase0 = sid * (L1_CHUNKS * CH) + hseg * (hseg_chunks * CH)
            _edge_loop(xl_hbm.at[h], xr_hbm.at[h], hseg_chunks, ebase0,
                       sv, tv, bufL, bufR, msg, den, attv, semL, semR,
                       semS, semD, acc, dacc)
            return 0

        lax.fori_loop(0, 4, hseg_body, 0)
        plsc.subcore_barrier()
        rb = sid * ROWS_PER_TILE
        pltpu.sync_copy(acc.at[pl.ds(rb, ROWS_PER_TILE)],
                        out_hbm.at[h].at[pl.ds(rb, ROWS_PER_TILE)])
        pltpu.sync_copy(dacc.at[pl.ds(rb, ROWS_PER_TILE)],
                        dout_hbm.at[h].at[pl.ds(rb, ROWS_PER_TILE)])
        plsc.subcore_barrier()


def _l2_body(xl_hbm, xr_hbm, s3_hbm, t3_hbm, att_hbm,
             out_hbm, dout_hbm,
             sv, tv, bufL, bufR, msg, den, attv, zbuf, dzbuf,
             semL, semR, semS, semD, acc, dacc):
    cid = lax.axis_index("c")
    sid = lax.axis_index("s")
    wid = cid * NT + sid
    _zero_scratch(zbuf, dzbuf)
    pltpu.sync_copy(att_hbm.at[0], attv)
    _zero_acc(sid, acc, dacc, zbuf, dzbuf)
    plsc.subcore_barrier()
    seg_chunks = L2_CHUNKS // 2

    def seg_body(seg, _):
        pltpu.sync_copy(s3_hbm.at[wid, pl.ds(seg * seg_chunks, seg_chunks)], sv)
        pltpu.sync_copy(t3_hbm.at[wid, pl.ds(seg * seg_chunks, seg_chunks)], tv)
        ebase0 = wid * (L2_CHUNKS * CH) + seg * (seg_chunks * CH)
        _edge_loop(xl_hbm, xr_hbm, seg_chunks, ebase0,
                   sv, tv, bufL, bufR, msg, den, attv, semL, semR,
                   semS, semD, acc, dacc)
        return 0

    lax.fori_loop(0, 2, seg_body, 0)
    plsc.subcore_barrier()
    rb = sid * ROWS_PER_TILE
    pltpu.sync_copy(acc.at[pl.ds(rb, ROWS_PER_TILE)],
                    out_hbm.at[cid].at[pl.ds(rb, ROWS_PER_TILE)])
    pltpu.sync_copy(dacc.at[pl.ds(rb, ROWS_PER_TILE)],
                    dout_hbm.at[cid].at[pl.ds(rb, ROWS_PER_TILE)])


def _sc_scratch(nchunks):
    return [
        pltpu.VMEM((nchunks, CH), jnp.int32),
        pltpu.VMEM((nchunks, CH), jnp.int32),
        pltpu.VMEM((2, CH, HID), jnp.float32),
        pltpu.VMEM((2, CH, HID), jnp.float32),
        pltpu.VMEM((2, CH, HID), jnp.float32),
        pltpu.VMEM((2, CH, 16), jnp.float32),
        pltpu.VMEM((HID,), jnp.float32),
        pltpu.VMEM((ZR, HID), jnp.float32),
        pltpu.VMEM((ZR, 16), jnp.float32),
        pltpu.SemaphoreType.DMA((2,)),
        pltpu.SemaphoreType.DMA((2,)),
        pltpu.SemaphoreType.DMA((2,)),
        pltpu.SemaphoreType.DMA((2,)),
        pltpu.VMEM_SHARED((NPAD, HID), jnp.float32),
        pltpu.VMEM_SHARED((NPAD, 16), jnp.float32),
    ]


def _sc_layer1(xl_t, xr_t, s3, t3, att):
    return pl.kernel(
        _l1_body,
        out_type=[
            jax.ShapeDtypeStruct((H, NPAD, HID), jnp.float32),
            jax.ShapeDtypeStruct((H, NPAD, 16), jnp.float32),
        ],
        mesh=_mesh,
        scratch_types=_sc_scratch(L1_CHUNKS // 4),
        compiler_params=pltpu.CompilerParams(
            needs_layout_passes=False, use_tc_tiling_on_sc=False),
    )(xl_t, xr_t, s3, t3, att)


def _sc_layer2(xl2, xr2, s3, t3, att):
    return pl.kernel(
        _l2_body,
        out_type=[
            jax.ShapeDtypeStruct((2, NPAD, HID), jnp.float32),
            jax.ShapeDtypeStruct((2, NPAD, 16), jnp.float32),
        ],
        mesh=_mesh,
        scratch_types=_sc_scratch(L2_CHUNKS // 2),
        compiler_params=pltpu.CompilerParams(
            needs_layout_passes=False, use_tc_tiling_on_sc=False),
    )(xl2, xr2, s3, t3, att)


def kernel(x, edge_idx, W_in, b_in, Wl1, Wr1, att1, bias1, Wl2, Wr2, att2, bias2, W_out, b_out):
    src, dst = edge_idx[0], edge_idx[1]
    loop = jnp.arange(N, dtype=jnp.int32)
    pad = jnp.zeros((EPAD - ETOT,), jnp.int32)
    s_ = jnp.concatenate([src, loop, pad])
    t_ = jnp.concatenate([dst, loop, pad])

    s3_l1 = s_.reshape(NT, L1_CHUNKS, CH)
    t3_l1 = t_.reshape(NT, L1_CHUNKS, CH)
    s3_l2 = s_.reshape(2 * NT, L2_CHUNKS, CH)
    t3_l2 = t_.reshape(2 * NT, L2_CHUNKS, CH)

    h0 = _input_proj(x, W_in, b_in)
    xl1, xr1 = _proj_l1(h0, Wl1, Wr1)
    out1, den1 = _sc_layer1(xl1, xr1, s3_l1, t3_l1, att1)
    xl2, xr2 = _proj_l2(out1, den1, bias1, Wl2, Wr2)
    out2, den2 = _sc_layer2(xl2, xr2, s3_l2, t3_l2, att2)
    return _output_head(out2, den2, bias2, W_out, b_out)


# speedup vs baseline: 1.1124x; 1.1124x over previous
import functools
import jax
import jax.numpy as jnp
from jax import lax
from jax.experimental import pallas as pl
from jax.experimental.pallas import tpu as pltpu
from jax.experimental.pallas import tpu_sc as plsc

N = 10000
E = 320000
D = 128
H = 4
HID = 128
C = 40

ETOT = E + N
EPAD = 331776
CH = 16
NT = 16
NPAD = 10240
ROWS_PER_TILE = NPAD // NT

L1_CHUNKS = EPAD // (NT * CH)
L2_CHUNKS = EPAD // (2 * NT * CH)

_mesh = plsc.VectorSubcoreMesh(core_axis_name="c", subcore_axis_name="s")


def _elu(v):
    return jnp.where(v > 0, v, jnp.exp(v) - 1.0)


def _tc1_body(x_ref, w_ref, b_ref, o_ref):
    v = jnp.dot(x_ref[...], w_ref[...], preferred_element_type=jnp.float32)
    o_ref[...] = _elu(v + b_ref[...][None, :])


def _input_proj(x, W_in, b_in):
    blk = 1000
    return pl.pallas_call(
        _tc1_body,
        grid=(N // blk,),
        in_specs=[
            pl.BlockSpec((blk, D), lambda i: (i, 0)),
            pl.BlockSpec((D, HID), lambda i: (0, 0)),
            pl.BlockSpec((HID,), lambda i: (0,)),
        ],
        out_specs=pl.BlockSpec((blk, HID), lambda i: (i, 0)),
        out_shape=jax.ShapeDtypeStruct((N, HID), jnp.float32),
    )(x, W_in, b_in)


def _tc2_body(h_ref, wl_ref, wr_ref, ol_ref, or_ref):
    blk = h_ref[...]
    ol_ref[0] = jnp.dot(blk, wl_ref[...], preferred_element_type=jnp.float32)
    or_ref[0] = jnp.dot(blk, wr_ref[...], preferred_element_type=jnp.float32)


def _proj_l1(h0, Wl1, Wr1):
    blk = 1000
    return pl.pallas_call(
        _tc2_body,
        grid=(N // blk, H),
        in_specs=[
            pl.BlockSpec((blk, HID), lambda i, h: (i, 0)),
            pl.BlockSpec((HID, HID), lambda i, h: (0, h)),
            pl.BlockSpec((HID, HID), lambda i, h: (0, h)),
        ],
        out_specs=[
            pl.BlockSpec((1, blk, HID), lambda i, h: (h, i, 0)),
            pl.BlockSpec((1, blk, HID), lambda i, h: (h, i, 0)),
        ],
        out_shape=[
            jax.ShapeDtypeStruct((H, N, HID), jnp.float32),
            jax.ShapeDtypeStruct((H, N, HID), jnp.float32),
        ],
    )(h0, Wl1, Wr1)


def _tc3_body(raw_ref, den_ref, b_ref, wl_ref, wr_ref, ol_ref, or_ref):
    h = pl.program_id(1)

    @pl.when(h == 0)
    def _():
        ol_ref[...] = jnp.zeros_like(ol_ref)
        or_ref[...] = jnp.zeros_like(or_ref)

    den = den_ref[0, :, 0:1] + 1e-16
    hp = _elu(raw_ref[0] / den + b_ref[0])
    ol_ref[...] += jnp.dot(hp, wl_ref[0], preferred_element_type=jnp.float32)
    or_ref[...] += jnp.dot(hp, wr_ref[0], preferred_element_type=jnp.float32)


def _proj_l2(out1, den1, bias1, Wl2, Wr2):
    blk = 1000
    return pl.pallas_call(
        _tc3_body,
        grid=(N // blk, H),
        in_specs=[
            pl.BlockSpec((1, blk, HID), lambda i, h: (h, i, 0)),
            pl.BlockSpec((1, blk, 16), lambda i, h: (h, i, 0)),
            pl.BlockSpec((1, 1, HID), lambda i, h: (h, 0, 0)),
            pl.BlockSpec((1, HID, HID), lambda i, h: (h, 0, 0)),
            pl.BlockSpec((1, HID, HID), lambda i, h: (h, 0, 0)),
        ],
        out_specs=[
            pl.BlockSpec((blk, HID), lambda i, h: (i, 0)),
            pl.BlockSpec((blk, HID), lambda i, h: (i, 0)),
        ],
        out_shape=[
            jax.ShapeDtypeStruct((N, HID), jnp.float32),
            jax.ShapeDtypeStruct((N, HID), jnp.float32),
        ],
    )(out1, den1, bias1.reshape(H, 1, HID), Wl2.reshape(H, HID, HID),
      Wr2.reshape(H, HID, HID))


def _tc4_body(raw_ref, den_ref, b2_ref, wo_ref, bo_ref, o_ref):
    raw = raw_ref[0] + raw_ref[1]
    den = den_ref[0, :, 0:1] + den_ref[1, :, 0:1] + 1e-16
    h2 = _elu(raw / den + b2_ref[...][None, :])
    logits = jnp.dot(h2, wo_ref[...], preferred_element_type=jnp.float32)
    logits = logits + bo_ref[...][None, :]
    m = jnp.max(logits, axis=1, keepdims=True)
    lse = m + jnp.log(jnp.sum(jnp.exp(logits - m), axis=1, keepdims=True))
    o_ref[...] = logits - lse


def _output_head(out2, den2, bias2, W_out, b_out):
    blk = 1000
    return pl.pallas_call(
        _tc4_body,
        grid=(N // blk,),
        in_specs=[
            pl.BlockSpec((2, blk, HID), lambda i: (0, i, 0)),
            pl.BlockSpec((2, blk, 16), lambda i: (0, i, 0)),
            pl.BlockSpec((HID,), lambda i: (0,)),
            pl.BlockSpec((HID, C), lambda i: (0, 0)),
            pl.BlockSpec((C,), lambda i: (0,)),
        ],
        out_specs=pl.BlockSpec((blk, C), lambda i: (i, 0)),
        out_shape=jax.ShapeDtypeStruct((N, C), jnp.float32),
    )(out2, den2, bias2, W_out, b_out)


ZR = 16


def _zero_scratch(zbuf, dzbuf):
    def zrow(i, _):
        for j in range(8):
            zbuf[i, pl.ds(j * 16, 16)] = jnp.zeros((16,), jnp.float32)
        dzbuf[i, :] = jnp.zeros((16,), jnp.float32)
        return 0

    lax.fori_loop(0, ZR, zrow, 0)


def _zero_acc(sid, acc, dacc, zbuf, dzbuf):
    base = sid * ROWS_PER_TILE
    for k in range(ROWS_PER_TILE // ZR):
        pltpu.sync_copy(zbuf, acc.at[pl.ds(base + k * ZR, ZR)])
        pltpu.sync_copy(dzbuf, dacc.at[pl.ds(base + k * ZR, ZR)])


def _edge_chunk(bufL, bufR, attv, msg, den, ebase):
    lane = lax.iota(jnp.int32, 16)
    att = [attv[pl.ds(j * 16, 16)] for j in range(8)]
    for g in range(CH // 16):
        alv = jnp.zeros((16,), jnp.float32)
        for el in range(16):
            e = g * 16 + el
            acc_a = jnp.zeros((16,), jnp.float32)
            for j in range(8):
                sl = pl.ds(j * 16, 16)
                a = bufL[e, sl] + bufR[e, sl]
                a = jnp.maximum(a, 0.2 * a)
                acc_a = acc_a + a * att[j]
            alpha_e = plsc.cumsum(acc_a)[15]
            alv = jnp.where(lane == el, alpha_e, alv)
        exv = jnp.exp(alv)
        exv = jnp.where(ebase + g * 16 + lane < ETOT, exv, 0.0)
        for el in range(16):
            e = g * 16 + el
            exe = exv[el]
            for j in range(8):
                sl = pl.ds(j * 16, 16)
                msg[e, sl] = exe * bufL[e, sl]
            den[e, :] = jnp.where(lane == 0, exe, 0.0)


def _edge_loop(xl_tab, xr_tab, nchunks, ebase0,
               sv, tv, bufL, bufR, msg, den, attv, semL, semR, semS, semD,
               acc, dacc):
    def issue(ci, b):
        pltpu.async_copy(xl_tab.at[sv.at[ci]], bufL.at[b], semL.at[b])
        pltpu.async_copy(xr_tab.at[tv.at[ci]], bufR.at[b], semR.at[b])

    def wait_scatter(mb):
        pltpu.make_async_copy(msg.at[mb], acc.at[tv.at[0]], semS.at[mb]).wait()
        pltpu.make_async_copy(den.at[mb], dacc.at[tv.at[0]], semD.at[mb]).wait()

    issue(0, 0)
    issue(1, 1)
    issue(2, 2)

    def quad(ci4, _):
        for b in range(4):
            ci = ci4 * 4 + b
            mb = b % 2
            pltpu.make_async_copy(xl_tab.at[sv.at[0]], bufL.at[b], semL.at[b]).wait()
            pltpu.make_async_copy(xr_tab.at[tv.at[0]], bufR.at[b], semR.at[b]).wait()

            @pl.when(ci + 3 < nchunks)
            def _():
                issue(ci + 3, (b + 3) % 4)

            if b < 2:
                @pl.when(ci4 > 0)
                def _():
                    wait_scatter(mb)
            else:
                wait_scatter(mb)

            _edge_chunk(bufL.at[b], bufR.at[b], attv, msg.at[mb], den.at[mb],
                        ebase0 + ci * CH)
            pltpu.async_copy(msg.at[mb], acc.at[tv.at[ci]], semS.at[mb], add=True)
            pltpu.async_copy(den.at[mb], dacc.at[tv.at[ci]], semD.at[mb], add=True)
        return 0

    lax.fori_loop(0, nchunks // 4, quad, 0)
    wait_scatter(0)
    wait_scatter(1)


def _l1_body(xl_hbm, xr_hbm, s3_hbm, t3_hbm, att_hbm,
             out_hbm, dout_hbm,
             sv, tv, bufL, bufR, msg, den, attv, zbuf, dzbuf,
             semL, semR, semS, semD, acc, dacc):
    cid = lax.axis_index("c")
    sid = lax.axis_index("s")
    _zero_scratch(zbuf, dzbuf)
    hseg_chunks = L1_CHUNKS // 4
    for hp in range(2):
        h = cid * 2 + hp
        pltpu.sync_copy(att_hbm.at[h], attv)
        _zero_acc(sid, acc, dacc, zbuf, dzbuf)
        plsc.subcore_barrier()

        def hseg_body(hseg, _):
            pltpu.sync_copy(s3_hbm.at[sid, pl.ds(hseg * hseg_chunks, hseg_chunks)], sv)
            pltpu.sync_copy(t3_hbm.at[sid, pl.ds(hseg * hseg_chunks, hseg_chunks)], tv)
            ebase0 = sid * (L1_CHUNKS * CH) + hseg * (hseg_chunks * CH)
            _edge_loop(xl_hbm.at[h], xr_hbm.at[h], hseg_chunks, ebase0,
                       sv, tv, bufL, bufR, msg, den, attv, semL, semR,
                       semS, semD, acc, dacc)
            return 0

        lax.fori_loop(0, 4, hseg_body, 0)
        plsc.subcore_barrier()
        rb = sid * ROWS_PER_TILE
        pltpu.sync_copy(acc.at[pl.ds(rb, ROWS_PER_TILE)],
                        out_hbm.at[h].at[pl.ds(rb, ROWS_PER_TILE)])
        pltpu.sync_copy(dacc.at[pl.ds(rb, ROWS_PER_TILE)],
                        dout_hbm.at[h].at[pl.ds(rb, ROWS_PER_TILE)])
        plsc.subcore_barrier()


def _l2_body(xl_hbm, xr_hbm, s3_hbm, t3_hbm, att_hbm,
             out_hbm, dout_hbm,
             sv, tv, bufL, bufR, msg, den, attv, zbuf, dzbuf,
             semL, semR, semS, semD, acc, dacc):
    cid = lax.axis_index("c")
    sid = lax.axis_index("s")
    wid = cid * NT + sid
    _zero_scratch(zbuf, dzbuf)
    pltpu.sync_copy(att_hbm.at[0], attv)
    _zero_acc(sid, acc, dacc, zbuf, dzbuf)
    plsc.subcore_barrier()
    seg_chunks = L2_CHUNKS // 2

    def seg_body(seg, _):
        pltpu.sync_copy(s3_hbm.at[wid, pl.ds(seg * seg_chunks, seg_chunks)], sv)
        pltpu.sync_copy(t3_hbm.at[wid, pl.ds(seg * seg_chunks, seg_chunks)], tv)
        ebase0 = wid * (L2_CHUNKS * CH) + seg * (seg_chunks * CH)
        _edge_loop(xl_hbm, xr_hbm, seg_chunks, ebase0,
                   sv, tv, bufL, bufR, msg, den, attv, semL, semR,
                   semS, semD, acc, dacc)
        return 0

    lax.fori_loop(0, 2, seg_body, 0)
    plsc.subcore_barrier()
    rb = sid * ROWS_PER_TILE
    pltpu.sync_copy(acc.at[pl.ds(rb, ROWS_PER_TILE)],
                    out_hbm.at[cid].at[pl.ds(rb, ROWS_PER_TILE)])
    pltpu.sync_copy(dacc.at[pl.ds(rb, ROWS_PER_TILE)],
                    dout_hbm.at[cid].at[pl.ds(rb, ROWS_PER_TILE)])


def _sc_scratch(nchunks):
    return [
        pltpu.VMEM((nchunks, CH), jnp.int32),
        pltpu.VMEM((nchunks, CH), jnp.int32),
        pltpu.VMEM((4, CH, HID), jnp.float32),
        pltpu.VMEM((4, CH, HID), jnp.float32),
        pltpu.VMEM((2, CH, HID), jnp.float32),
        pltpu.VMEM((2, CH, 16), jnp.float32),
        pltpu.VMEM((HID,), jnp.float32),
        pltpu.VMEM((ZR, HID), jnp.float32),
        pltpu.VMEM((ZR, 16), jnp.float32),
        pltpu.SemaphoreType.DMA((4,)),
        pltpu.SemaphoreType.DMA((4,)),
        pltpu.SemaphoreType.DMA((2,)),
        pltpu.SemaphoreType.DMA((2,)),
        pltpu.VMEM_SHARED((NPAD, HID), jnp.float32),
        pltpu.VMEM_SHARED((NPAD, 16), jnp.float32),
    ]


def _sc_layer1(xl_t, xr_t, s3, t3, att):
    return pl.kernel(
        _l1_body,
        out_type=[
            jax.ShapeDtypeStruct((H, NPAD, HID), jnp.float32),
            jax.ShapeDtypeStruct((H, NPAD, 16), jnp.float32),
        ],
        mesh=_mesh,
        scratch_types=_sc_scratch(L1_CHUNKS // 4),
        compiler_params=pltpu.CompilerParams(
            needs_layout_passes=False, use_tc_tiling_on_sc=False),
    )(xl_t, xr_t, s3, t3, att)


def _sc_layer2(xl2, xr2, s3, t3, att):
    return pl.kernel(
        _l2_body,
        out_type=[
            jax.ShapeDtypeStruct((2, NPAD, HID), jnp.float32),
            jax.ShapeDtypeStruct((2, NPAD, 16), jnp.float32),
        ],
        mesh=_mesh,
        scratch_types=_sc_scratch(L2_CHUNKS // 2),
        compiler_params=pltpu.CompilerParams(
            needs_layout_passes=False, use_tc_tiling_on_sc=False),
    )(xl2, xr2, s3, t3, att)


def kernel(x, edge_idx, W_in, b_in, Wl1, Wr1, att1, bias1, Wl2, Wr2, att2, bias2, W_out, b_out):
    src, dst = edge_idx[0], edge_idx[1]
    loop = jnp.arange(N, dtype=jnp.int32)
    pad = jnp.zeros((EPAD - ETOT,), jnp.int32)
    s_ = jnp.concatenate([src, loop, pad])
    t_ = jnp.concatenate([dst, loop, pad])

    s3_l1 = s_.reshape(NT, L1_CHUNKS, CH)
    t3_l1 = t_.reshape(NT, L1_CHUNKS, CH)
    s3_l2 = s_.reshape(2 * NT, L2_CHUNKS, CH)
    t3_l2 = t_.reshape(2 * NT, L2_CHUNKS, CH)

    h0 = _input_proj(x, W_in, b_in)
    xl1, xr1 = _proj_l1(h0, Wl1, Wr1)
    out1, den1 = _sc_layer1(xl1, xr1, s3_l1, t3_l1, att1)
    xl2, xr2 = _proj_l2(out1, den1, bias1, Wl2, Wr2)
    out2, den2 = _sc_layer2(xl2, xr2, s3_l2, t3_l2, att2)
    return _output_head(out2, den2, bias2, W_out, b_out)
